# Initial kernel scaffold; baseline (speedup 1.0000x reference)
import jax
import jax.numpy as jnp
from jax.experimental import pallas as pl


def kernel(input_tensor, table, hash_a, hash_b):
    raise NotImplementedError("write your pallas kernel here")


# speedup vs baseline: 1280.2622x; 1280.2622x over previous
import jax
import jax.numpy as jnp
from jax import lax
from jax.experimental import pallas as pl
from jax.experimental.pallas import tpu as pltpu
from jax.experimental.pallas import tpu_sc as plsc

P31 = (1 << 31) - 1
SIZE = 1000000
CHUNK = 16
NH = 4
B, F = 16384, 26
N_IDS = B * F
NC, NS = 2, 16
NW = NC * NS
IDS_PER_W = N_IDS // NW
C = 256
NCH = IDS_PER_W // C
NSL = NH * C
SH_ROWS128 = 7816
SH_ROWS16 = SH_ROWS128 * 8
ARR_ROWS = 7824


def _tc_shift_body(tbl_ref, out_ref):
    o = pl.program_id(0)
    a = tbl_ref[0:SH_ROWS128, :]
    b = tbl_ref[1:SH_ROWS128 + 1, :]
    sh = (jnp.int32(128) - o) & jnp.int32(127)
    ra = pltpu.roll(a, sh, 1)
    rb = pltpu.roll(b, sh, 1)
    lane = lax.broadcasted_iota(jnp.int32, (SH_ROWS128, 128), 1)
    out_ref[0] = jnp.where(lane < jnp.int32(128) - o, ra, rb)


@jax.jit
def _build_shifted(arr):
    return pl.pallas_call(
        _tc_shift_body,
        grid=(16,),
        in_specs=[pl.BlockSpec(
            (ARR_ROWS, 128), lambda o: (jnp.int32(0), jnp.int32(0)))],
        out_specs=pl.BlockSpec(
            (1, SH_ROWS128, 128),
            lambda o: (o, jnp.int32(0), jnp.int32(0))),
        out_shape=jax.ShapeDtypeStruct((16, SH_ROWS128, 128), jnp.float32),
    )(arr)


def _wred(r, c31):
    return (r & P31) + lax.shift_right_logical(r, c31)


def _csub(r):
    return jnp.where(r >= P31, r - P31, r)


def _shm16(hi_coef, v):
    prod = hi_coef * v
    return ((prod & ((1 << 15) - 1)) << 16) + (prod >> 15)


def _sc_body(ids_hbm, tbl_hbm, par_hbm, out_hbm,
             ids_v, lut1_v, lut0_v, idx_v, rows_v, par_v, gsem, wsem):
    c = lax.axis_index("c")
    s = lax.axis_index("s")
    wid = s * jnp.int32(NC) + c
    id_base = wid * jnp.int32(IDS_PER_W)
    lanes = lax.iota(jnp.int32, 16)
    c31 = jnp.full((16,), 31, jnp.int32)

    pltpu.sync_copy(par_hbm, par_v)
    pltpu.sync_copy(ids_hbm.at[pl.ds(id_base, IDS_PER_W)], ids_v)

    def lut_body(m, _):
        v = lanes + m * jnp.int32(16)
        for j in range(NH):
            Ah = par_v[j]
            Al = par_v[NH + j]
            ah = par_v[2 * NH + j]
            al = par_v[3 * NH + j]
            bb = par_v[4 * NH + j]
            h1 = _csub(_wred(_shm16(Ah, v) + Al * v, c31))
            lut1_v[pl.ds(m * jnp.int32(16) + jnp.int32(j * 1024), 16)] = h1
            t = _wred(_shm16(ah, v) + al * v, c31)
            h0 = _csub(_wred(t + bb, c31))
            lut0_v[pl.ds(m * jnp.int32(16) + jnp.int32(j * 1024), 16)] = h0
        return jnp.int32(0)

    lax.fori_loop(jnp.int32(0), jnp.int32(1024 // 16), lut_body, jnp.int32(0))

    lanes4 = lanes * jnp.int32(NH)

    def hash_chunk(g, po):
        for gg in range(C // 16):
            x = ids_v[pl.ds(g * jnp.int32(C) + jnp.int32(gg * 16), 16)]
            xh = x >> 10
            xl = x & 1023
            for j in range(NH):
                g1 = plsc.load_gather(lut1_v, [xh + j * 1024])
                g0 = plsc.load_gather(lut0_v, [xl + j * 1024])
                h = _csub(_wred(g1 + g0, c31))
                q = (h.astype(jnp.float32) * jnp.float32(1.0 / SIZE)
                     ).astype(jnp.int32)
                r = h - q * SIZE
                r = r + ((r >> 31) & SIZE)
                t = r - SIZE
                r = t + ((t >> 31) & SIZE)
                row = (r >> 4) + (r & 15) * jnp.int32(SH_ROWS16)
                pos = lanes4 + (po + jnp.int32(gg * 64 + j))
                plsc.store_scatter(idx_v, [pos], row)

    def wb_copy(cix, po):
        wb_base = (id_base + cix * jnp.int32(C)) * jnp.int32(NH)
        return pltpu.make_async_copy(rows_v.at[pl.ds(po, NSL)],
                                     out_hbm.at[pl.ds(wb_base, NSL)], wsem)

    hash_chunk(jnp.int32(0), jnp.int32(0))

    def chunk_iter(g, _):
        p = g & jnp.int32(1)
        po = p * jnp.int32(NSL)
        qo = (jnp.int32(1) - p) * jnp.int32(NSL)

        @pl.when(g >= jnp.int32(2))
        def _():
            wb_copy(g - jnp.int32(2), po).wait()

        gathers = [
            pltpu.async_copy(
                tbl_hbm.at[idx_v.at[pl.ds(po + jnp.int32(k * 128), 128)]],
                rows_v.at[pl.ds(po + jnp.int32(k * 128), 128)], gsem)
            for k in range(NSL // 128)
        ]

        @pl.when(g < jnp.int32(NCH - 1))
        def _():
            hash_chunk(g + jnp.int32(1), qo)

        for cp in gathers:
            cp.wait()
        wb_copy(g, po).start()
        return jnp.int32(0)

    lax.fori_loop(jnp.int32(0), jnp.int32(NCH), chunk_iter, jnp.int32(0))
    wb_copy(jnp.int32(NCH - 2), jnp.int32(0)).wait()
    wb_copy(jnp.int32(NCH - 1), jnp.int32(NSL)).wait()


@jax.jit
def _robe_sc(ids, tbl_rows, params):
    mesh = plsc.VectorSubcoreMesh(core_axis_name="c", subcore_axis_name="s",
                                  num_cores=NC, num_subcores=NS)
    fn = pl.kernel(
        _sc_body,
        out_type=jax.ShapeDtypeStruct((N_IDS * NH, CHUNK), jnp.float32),
        mesh=mesh,
        scratch_types=[
            pltpu.VMEM((IDS_PER_W,), jnp.int32),
            pltpu.VMEM((NH * 1024,), jnp.int32),
            pltpu.VMEM((NH * 1024,), jnp.int32),
            pltpu.VMEM((2 * NSL,), jnp.int32),
            pltpu.VMEM((2 * NSL, CHUNK), jnp.float32),
            pltpu.VMEM((5 * NH, 16), jnp.int32),
            pltpu.SemaphoreType.DMA,
            pltpu.SemaphoreType.DMA,
        ],
        compiler_params=pltpu.CompilerParams(needs_layout_passes=False,
                                             use_tc_tiling_on_sc=False),
    )
    return fn(ids, tbl_rows, params)


def kernel(input_tensor, table, hash_a, hash_b):
    ids = input_tensor.reshape(-1).astype(jnp.int32)
    tbl_ext = jnp.concatenate([table, table[:ARR_ROWS * 128 - SIZE]])
    shifted = _build_shifted(tbl_ext.reshape(ARR_ROWS, 128))
    tbl_rows = shifted.reshape(16 * SH_ROWS16, CHUNK)
    A = (hash_a * 1024) % P31
    rows = [A >> 16, A & 0xFFFF, hash_a >> 16, hash_a & 0xFFFF, hash_b]
    params = jnp.broadcast_to(
        jnp.concatenate(rows).astype(jnp.int32)[:, None], (5 * NH, 16))
    out = _robe_sc(ids, tbl_rows, params)
    return out.reshape(B, F, NH * CHUNK)
